# baseline (speedup 1.0000x reference)
import jax
import jax.numpy as jnp
from jax.experimental import pallas as pl


def _pos_kernel(row_ref, col_ref, out_ref):
    half = pl.program_id(1)

    @pl.when(half == 0)
    def _():
        t = jnp.transpose(col_ref[...], (1, 0))
        out_ref[0] = jnp.broadcast_to(t[:, None, :], (256, 64, 64))

    @pl.when(half == 1)
    def _():
        t = jnp.transpose(row_ref[...], (1, 0))
        out_ref[0] = jnp.broadcast_to(t[:, :, None], (256, 64, 64))


def kernel(mask, row_embed, col_embed):
    b = mask.shape[0]
    h, w = mask.shape[-2], mask.shape[-1]
    f = col_embed.shape[-1]

    out = pl.pallas_call(
        _pos_kernel,
        grid=(b, 2),
        in_specs=[
            pl.BlockSpec((h, f), lambda i, j: (0, 0)),
            pl.BlockSpec((w, f), lambda i, j: (0, 0)),
        ],
        out_specs=pl.BlockSpec((1, f, h, w), lambda i, j: (i, j, 0, 0)),
        out_shape=jax.ShapeDtypeStruct((b, 2 * f, h, w), jnp.float32),
    )(row_embed, col_embed)
    return out


# speedup vs baseline: 1.6380x; 1.6380x over previous
import jax
import jax.numpy as jnp
from jax.experimental import pallas as pl
from jax.experimental.pallas import tpu as pltpu


def _pos_kernel(row_ref, col_ref, out_ref, scratch, sems):
    b = out_ref.shape[0]
    f = col_ref.shape[1]
    h = row_ref.shape[0]
    w = col_ref.shape[0]

    tcol = jnp.transpose(col_ref[...], (1, 0))
    scratch[0:f, :] = jnp.broadcast_to(tcol[:, None, :], (f, h, w)).reshape(f, h * w)
    for i in range(b):
        pltpu.make_async_copy(
            scratch.at[pl.ds(0, f)], out_ref.at[i, pl.ds(0, f)], sems.at[i, 0]
        ).start()

    trow = jnp.transpose(row_ref[...], (1, 0))
    scratch[f : 2 * f, :] = jnp.broadcast_to(trow[:, :, None], (f, h, w)).reshape(
        f, h * w
    )
    for i in range(b):
        pltpu.make_async_copy(
            scratch.at[pl.ds(f, f)], out_ref.at[i, pl.ds(f, f)], sems.at[i, 1]
        ).start()

    for i in range(b):
        pltpu.make_async_copy(
            scratch.at[pl.ds(0, f)], out_ref.at[i, pl.ds(0, f)], sems.at[i, 0]
        ).wait()
        pltpu.make_async_copy(
            scratch.at[pl.ds(f, f)], out_ref.at[i, pl.ds(f, f)], sems.at[i, 1]
        ).wait()


def kernel(mask, row_embed, col_embed):
    b = mask.shape[0]
    h, w = mask.shape[-2], mask.shape[-1]
    f = col_embed.shape[-1]

    out = pl.pallas_call(
        _pos_kernel,
        in_specs=[
            pl.BlockSpec(memory_space=pltpu.MemorySpace.VMEM),
            pl.BlockSpec(memory_space=pltpu.MemorySpace.VMEM),
        ],
        out_specs=pl.BlockSpec(memory_space=pltpu.MemorySpace.HBM),
        out_shape=jax.ShapeDtypeStruct((b, 2 * f, h * w), jnp.float32),
        scratch_shapes=[
            pltpu.VMEM((2 * f, h * w), jnp.float32),
            pltpu.SemaphoreType.DMA((b, 2)),
        ],
    )(row_embed, col_embed)
    return out.reshape(b, 2 * f, h, w)
